# Initial kernel scaffold; baseline (speedup 1.0000x reference)
import jax
import jax.numpy as jnp
from jax.experimental import pallas as pl


def kernel(sims, img_ids):
    raise NotImplementedError("write your pallas kernel here")


# speedup vs baseline: 32.1821x; 32.1821x over previous
import jax
import jax.numpy as jnp
import numpy as np
from jax.experimental import pallas as pl
from jax.experimental.pallas import tpu as pltpu

_D_CONST = 1024.0
_CUT_OFF = 0.1
_NONZERO_LOSS_CUTOFF = 1.9
_MARGIN = 0.5
_N = 1024
_BLK = 8


def _threefry_block(x0, x1, k0, k1, k2):
    rotations = ((13, 15, 26, 6), (17, 29, 16, 24))
    ks = (k0, k1, k2)
    x0 = x0 + ks[0]
    x1 = x1 + ks[1]
    for i in range(5):
        for r in rotations[i % 2]:
            x0 = x0 + x1
            x1 = (x1 << jnp.uint32(r)) | (x1 >> jnp.uint32(32 - r))
            x1 = x0 ^ x1
        x0 = x0 + ks[(i + 1) % 3]
        x1 = x1 + ks[(i + 2) % 3] + jnp.uint32(i + 1)
    return x0, x1


def _loss_kernel(ids_smem, keys_smem, sims_ref, simsT_ref, idsv_ref, idsc_ref,
                 out_ref, rank_ref):
    tiny = jnp.float32(np.finfo(np.float32).tiny)
    col_i = jax.lax.broadcasted_iota(jnp.int32, (1, _N), 1)
    col_u = jax.lax.broadcasted_iota(jnp.int32, (_BLK, _N), 1).astype(jnp.uint32)
    row8 = jax.lax.broadcasted_iota(jnp.int32, (_BLK, 1), 0)

    idsv = idsv_ref[...]

    maskM = (jnp.broadcast_to(idsc_ref[...], (_N, _N)) ==
             jnp.broadcast_to(idsv, (_N, _N))).astype(jnp.float32)
    tri = (jax.lax.broadcasted_iota(jnp.int32, (_N, _N), 0) <=
           jax.lax.broadcasted_iota(jnp.int32, (_N, _N), 1)).astype(jnp.float32)
    rank_ref[...] = jax.lax.dot_general(
        maskM, tri, (((1,), (0,)), ((), ())),
        preferred_element_type=jnp.float32) - 1.0

    bins = jax.lax.broadcasted_iota(jnp.int32, (128, _N), 0)
    eq = (bins == jnp.broadcast_to(idsv, (128, _N))).astype(jnp.int32)
    counts = jnp.sum(eq, axis=1, keepdims=True)
    num_pos = jnp.sum(counts * counts)
    half = (num_pos * _N).astype(jnp.uint32) // jnp.uint32(2)

    def run_direction(sref, key_slot, acc0):
        k0 = keys_smem[key_slot, 0].astype(jnp.uint32)
        k1 = keys_smem[key_slot, 1].astype(jnp.uint32)
        k2 = keys_smem[key_slot, 2].astype(jnp.uint32)

        def row_body(i, carry):
            off, acc = carry
            id_i = ids_smem[i]
            mask = idsv == id_i
            maskf = mask.astype(jnp.float32)
            negf = 1.0 - maskf
            s = sref[pl.ds(i, 1), :]
            dist = jnp.sqrt(2.0 - 2.0 * s)
            dist = jnp.maximum(dist, _CUT_OFF)
            lw = (2.0 - _D_CONST) * jnp.log(dist) - (_D_CONST - 3.0) / 2.0 * jnp.log(
                1.0 - 0.25 * (dist * dist))
            inf_nan = jnp.isinf(lw) | jnp.isnan(lw)
            lw = lw * negf
            lw = jnp.where(inf_nan, 0.0, lw)
            mx = jnp.max(lw)
            w = jnp.exp(lw - mx)
            w = w * (negf * (dist < _NONZERO_LOSS_CUTOFF).astype(jnp.float32))
            w = w / (jnp.sum(w) + 1e-20)
            logits = jnp.log(w + 1e-20)

            c = jnp.sum(mask.astype(jnp.int32))
            ranks = rank_ref[pl.ds(i, 1), :]
            s_b = jnp.broadcast_to(s, (_BLK, _N))
            nb = (c + (_BLK - 1)) // _BLK

            def blk(b, acc2):
                kk = b * _BLK + row8
                t = off + kk
                g = t.astype(jnp.uint32) * jnp.uint32(_N) + col_u
                lo = g < half
                x0 = jnp.where(lo, g, g - half)
                x1 = x0 + half
                h0, h1 = _threefry_block(x0, x1, k0, k1, k2)
                bits = jnp.where(lo, h0, h1)
                fb = (bits >> jnp.uint32(9)) | jnp.uint32(0x3F800000)
                f = jax.lax.bitcast_convert_type(fb, jnp.float32) - jnp.float32(1.0)
                u = jnp.maximum(tiny, f * (jnp.float32(1.0) - tiny) + tiny)
                gum = -jnp.log(-jnp.log(u))
                score = gum + logits
                smax = jnp.max(score, axis=1, keepdims=True)
                neg = jnp.min(jnp.where(score == smax, col_i, _N + 1),
                              axis=1, keepdims=True)
                s_an = jnp.sum(jnp.where(col_i == neg, s_b, 0.0),
                               axis=1, keepdims=True)
                sel_p = (ranks == kk.astype(jnp.float32)) & mask
                s_ap = jnp.sum(jnp.where(sel_p, s_b, 0.0),
                               axis=1, keepdims=True)
                valid = (kk < c).astype(jnp.float32)
                contrib = jnp.sum(valid * jnp.maximum(_MARGIN + s_an - s_ap, 0.0))
                return acc2 + contrib

            acc = jax.lax.fori_loop(0, nb, blk, acc)
            return off + c, acc

        _, acc = jax.lax.fori_loop(0, _N, row_body, (jnp.int32(0), acc0))
        return acc

    acc = run_direction(sims_ref, 0, jnp.float32(0.0))
    acc = run_direction(simsT_ref, 1, acc)
    out_ref[0, 0] = acc


def kernel(sims, img_ids):
    sims = sims.astype(jnp.float32)
    ids = img_ids.astype(jnp.int32)

    skey = jax.random.key(42)
    k1, k2 = jax.random.split(skey)
    kd1 = jax.random.key_data(k1).astype(jnp.uint32)
    kd2 = jax.random.key_data(k2).astype(jnp.uint32)
    magic = jnp.uint32(0x1BD11BDA)
    keys = jnp.stack([
        jnp.stack([kd1[0], kd1[1], kd1[0] ^ kd1[1] ^ magic]),
        jnp.stack([kd2[0], kd2[1], kd2[0] ^ kd2[1] ^ magic]),
    ]).astype(jnp.int32)

    out = pl.pallas_call(
        _loss_kernel,
        in_specs=[
            pl.BlockSpec(memory_space=pltpu.SMEM),
            pl.BlockSpec(memory_space=pltpu.SMEM),
            pl.BlockSpec(memory_space=pltpu.VMEM),
            pl.BlockSpec(memory_space=pltpu.VMEM),
            pl.BlockSpec(memory_space=pltpu.VMEM),
            pl.BlockSpec(memory_space=pltpu.VMEM),
        ],
        out_specs=pl.BlockSpec(memory_space=pltpu.SMEM),
        out_shape=jax.ShapeDtypeStruct((1, 1), jnp.float32),
        scratch_shapes=[pltpu.VMEM((_N, _N), jnp.float32)],
    )(ids, keys, sims, sims.T, ids.reshape(1, _N), ids.reshape(_N, 1))
    return out.reshape(())


# speedup vs baseline: 50.2272x; 1.5607x over previous
import jax
import jax.numpy as jnp
import numpy as np
from jax.experimental import pallas as pl
from jax.experimental.pallas import tpu as pltpu

_D_CONST = 1024.0
_CUT_OFF = 0.1
_NONZERO_LOSS_CUTOFF = 1.9
_MARGIN = 0.5
_N = 1024
_BLK = 16


def _threefry_block(x0, x1, k0, k1, k2):
    rotations = ((13, 15, 26, 6), (17, 29, 16, 24))
    ks = (k0, k1, k2)
    x0 = x0 + ks[0]
    x1 = x1 + ks[1]
    for i in range(5):
        for r in rotations[i % 2]:
            x0 = x0 + x1
            x1 = (x1 << jnp.uint32(r)) | (x1 >> jnp.uint32(32 - r))
            x1 = x0 ^ x1
        x0 = x0 + ks[(i + 1) % 3]
        x1 = x1 + ks[(i + 2) % 3] + jnp.uint32(i + 1)
    return x0, x1


def _logits_of(s, negf):
    dist = jnp.sqrt(2.0 - 2.0 * s)
    dist = jnp.maximum(dist, _CUT_OFF)
    lw = (2.0 - _D_CONST) * jnp.log(dist) - (_D_CONST - 3.0) / 2.0 * jnp.log(
        1.0 - 0.25 * (dist * dist))
    inf_nan = jnp.isinf(lw) | jnp.isnan(lw)
    lw = lw * negf
    lw = jnp.where(inf_nan, 0.0, lw)
    mx = jnp.max(lw, axis=1, keepdims=True)
    w = jnp.exp(lw - mx)
    w = w * (negf * (dist < _NONZERO_LOSS_CUTOFF).astype(jnp.float32))
    w = w / (jnp.sum(w, axis=1, keepdims=True) + 1e-20)
    return jnp.log(w + 1e-20)


def _prepass_kernel(sims_ref, simsT_ref, idsv_ref, idsc_ref, tri_ref,
                    logits1_ref, logits2_ref, rankm_ref, counts_ref):
    idsv = idsv_ref[...]
    maskM = jnp.broadcast_to(idsc_ref[...], (_N, _N)) == jnp.broadcast_to(
        idsv, (_N, _N))
    maskf = maskM.astype(jnp.float32)
    negf = 1.0 - maskf
    logits1_ref[...] = _logits_of(sims_ref[...], negf)
    logits2_ref[...] = _logits_of(simsT_ref[...], negf)
    ranks = jax.lax.dot_general(
        maskf, tri_ref[...], (((1,), (0,)), ((), ())),
        preferred_element_type=jnp.float32) - 1.0
    rankm_ref[...] = jnp.where(maskM, ranks, -2.0)
    counts_ref[...] = (ranks[:, _N - 1:_N] + 1.0).astype(jnp.int32)


def _sampler_kernel(counts_smem, keys_smem, sims_ref, simsT_ref,
                    logits1_ref, logits2_ref, rankm_ref, idsv_ref, out_ref):
    tiny = jnp.float32(np.finfo(np.float32).tiny)
    col_i = jax.lax.broadcasted_iota(jnp.int32, (1, _N), 1)
    col_u = jax.lax.broadcasted_iota(jnp.int32, (_BLK, _N), 1).astype(jnp.uint32)
    rowb = jax.lax.broadcasted_iota(jnp.int32, (_BLK, 1), 0)

    idsv = idsv_ref[...]
    bins = jax.lax.broadcasted_iota(jnp.int32, (128, _N), 0)
    eq = (bins == jnp.broadcast_to(idsv, (128, _N))).astype(jnp.int32)
    cnts = jnp.sum(eq, axis=1, keepdims=True)
    num_pos = jnp.sum(cnts * cnts)
    half = (num_pos * _N).astype(jnp.uint32) // jnp.uint32(2)

    def run_direction(sref, lref, key_slot, acc0):
        k0 = keys_smem[key_slot, 0].astype(jnp.uint32)
        k1 = keys_smem[key_slot, 1].astype(jnp.uint32)
        k2 = keys_smem[key_slot, 2].astype(jnp.uint32)

        def row_body(i, carry):
            off, acc = carry
            c = counts_smem[i]
            s = sref[pl.ds(i, 1), :]
            logits = lref[pl.ds(i, 1), :]
            rankm = rankm_ref[pl.ds(i, 1), :]
            s_b = jnp.broadcast_to(s, (_BLK, _N))
            nb = (c + (_BLK - 1)) // _BLK

            def blk(b, acc2):
                kk = b * _BLK + rowb
                t = off + kk
                g = t.astype(jnp.uint32) * jnp.uint32(_N) + col_u
                lo = g < half
                x0 = jnp.where(lo, g, g - half)
                x1 = x0 + half
                h0, h1 = _threefry_block(x0, x1, k0, k1, k2)
                bits = jnp.where(lo, h0, h1)
                fb = (bits >> jnp.uint32(9)) | jnp.uint32(0x3F800000)
                f = jax.lax.bitcast_convert_type(fb, jnp.float32) - jnp.float32(1.0)
                u = jnp.maximum(tiny, f * (jnp.float32(1.0) - tiny) + tiny)
                gum = -jnp.log(-jnp.log(u))
                score = gum + logits
                smax = jnp.max(score, axis=1, keepdims=True)
                neg = jnp.min(jnp.where(score == smax, col_i, _N + 1),
                              axis=1, keepdims=True)
                sel_an = col_i == neg
                sel_ap = rankm == kk.astype(jnp.float32)
                diff = jnp.sum(jnp.where(sel_an, s_b, 0.0) -
                               jnp.where(sel_ap, s_b, 0.0),
                               axis=1, keepdims=True)
                valid = (kk < c).astype(jnp.float32)
                contrib = jnp.sum(valid * jnp.maximum(_MARGIN + diff, 0.0))
                return acc2 + contrib

            acc = jax.lax.fori_loop(0, nb, blk, acc)
            return off + c, acc

        _, acc = jax.lax.fori_loop(0, _N, row_body, (jnp.int32(0), acc0))
        return acc

    acc = run_direction(sims_ref, logits1_ref, 0, jnp.float32(0.0))
    acc = run_direction(simsT_ref, logits2_ref, 1, acc)
    out_ref[0, 0] = acc


def kernel(sims, img_ids):
    sims = sims.astype(jnp.float32)
    simsT = sims.T
    ids = img_ids.astype(jnp.int32)
    tri = jnp.triu(jnp.ones((_N, _N), jnp.float32))

    skey = jax.random.key(42)
    k1, k2 = jax.random.split(skey)
    kd1 = jax.random.key_data(k1).astype(jnp.uint32)
    kd2 = jax.random.key_data(k2).astype(jnp.uint32)
    magic = jnp.uint32(0x1BD11BDA)
    keys = jnp.stack([
        jnp.stack([kd1[0], kd1[1], kd1[0] ^ kd1[1] ^ magic]),
        jnp.stack([kd2[0], kd2[1], kd2[0] ^ kd2[1] ^ magic]),
    ]).astype(jnp.int32)

    logits1, logits2, rankm, counts = pl.pallas_call(
        _prepass_kernel,
        in_specs=[pl.BlockSpec(memory_space=pltpu.VMEM)] * 5,
        out_specs=[pl.BlockSpec(memory_space=pltpu.VMEM)] * 4,
        out_shape=[
            jax.ShapeDtypeStruct((_N, _N), jnp.float32),
            jax.ShapeDtypeStruct((_N, _N), jnp.float32),
            jax.ShapeDtypeStruct((_N, _N), jnp.float32),
            jax.ShapeDtypeStruct((_N, 1), jnp.int32),
        ],
    )(sims, simsT, ids.reshape(1, _N), ids.reshape(_N, 1), tri)

    out = pl.pallas_call(
        _sampler_kernel,
        in_specs=[
            pl.BlockSpec(memory_space=pltpu.SMEM),
            pl.BlockSpec(memory_space=pltpu.SMEM),
            pl.BlockSpec(memory_space=pltpu.VMEM),
            pl.BlockSpec(memory_space=pltpu.VMEM),
            pl.BlockSpec(memory_space=pltpu.VMEM),
            pl.BlockSpec(memory_space=pltpu.VMEM),
            pl.BlockSpec(memory_space=pltpu.VMEM),
            pl.BlockSpec(memory_space=pltpu.VMEM),
        ],
        out_specs=pl.BlockSpec(memory_space=pltpu.SMEM),
        out_shape=jax.ShapeDtypeStruct((1, 1), jnp.float32),
    )(counts.reshape(_N), keys, sims, simsT, logits1, logits2, rankm,
      ids.reshape(1, _N))
    return out.reshape(())
